# baseline (speedup 1.0000x reference)
import functools

import jax
import jax.numpy as jnp
from jax import lax
from jax.experimental import pallas as pl
from jax.experimental.pallas import tpu as pltpu
from jax.experimental.pallas import tpu_sc as plsc

N = 10000
E = 320000
D = 128
H = 128
OUT = 128
EPS = 1e-5

NPAD = 10240
CH = 128
ROWS = 2560
NSUB = 16
NCORE = 2
RPT = ROWS // (NSUB * NCORE)
PER_SUB = NPAD // NSUB
PAD_DST = NPAD - 2


def _sc_mesh():
    return plsc.VectorSubcoreMesh(core_axis_name="c", subcore_axis_name="s")


def _sc_degree(dst2d):
    @functools.partial(
        pl.kernel,
        out_type=jax.ShapeDtypeStruct((NCORE, NPAD), jnp.float32),
        mesh=_sc_mesh(),
        scratch_types=[
            pltpu.VMEM((1, CH), jnp.int32),
            pltpu.VMEM((CH,), jnp.float32),
            pltpu.VMEM((PER_SUB,), jnp.float32),
            pltpu.VMEM_SHARED((NPAD,), jnp.float32),
        ],
    )
    def k(dst_hbm, degp_hbm, idx_v, ones_v, zbuf_v, acc_sh):
        c = lax.axis_index("c")
        s = lax.axis_index("s")
        wid = c * NSUB + s
        for i in range(CH // 16):
            ones_v[pl.ds(i * 16, 16)] = jnp.ones((16,), jnp.float32)

        @pl.loop(0, PER_SUB // 16)
        def _(i):
            zbuf_v[pl.ds(i * 16, 16)] = jnp.zeros((16,), jnp.float32)

        pltpu.sync_copy(zbuf_v, acc_sh.at[pl.ds(s * PER_SUB, PER_SUB)])
        plsc.subcore_barrier()

        base = wid * RPT

        @pl.loop(0, RPT)
        def _(kk):
            pltpu.sync_copy(dst_hbm.at[pl.ds(base + kk, 1)], idx_v)
            pltpu.sync_copy(ones_v, acc_sh.at[idx_v.at[0]], add=True)

        plsc.subcore_barrier()
        pltpu.sync_copy(acc_sh.at[pl.ds(s * PER_SUB, PER_SUB)], zbuf_v)
        pltpu.sync_copy(zbuf_v, degp_hbm.at[c, pl.ds(s * PER_SUB, PER_SUB)])

    return k(dst2d)


NB = 2
IDXH = RPT // 2


def _sc_aggregate(g, src2d, dst2d):
    @functools.partial(
        pl.kernel,
        out_type=jax.ShapeDtypeStruct((NCORE, NPAD, D), jnp.float32),
        mesh=_sc_mesh(),
        scratch_types=[
            pltpu.VMEM((IDXH, CH), jnp.int32),
            pltpu.VMEM((IDXH, CH), jnp.int32),
        ]
        + [pltpu.VMEM((CH, D), jnp.float32) for _ in range(NB)]
        + [pltpu.VMEM_SHARED((NPAD, D), jnp.float32)]
        + [pltpu.SemaphoreType.DMA for _ in range(2 * NB)],
    )
    def k(g_hbm, src_hbm, dst_hbm, aggp_hbm, sidx_v, didx_v, *rest):
        rows = rest[:NB]
        acc_sh = rest[NB]
        gsem = rest[NB + 1:2 * NB + 1]
        ssem = rest[2 * NB + 1:3 * NB + 1]
        c = lax.axis_index("c")
        s = lax.axis_index("s")
        wid = c * NSUB + s
        base = wid * RPT

        with jax.named_scope("agg_zero"):
            @pl.loop(0, CH)
            def _(r):
                for kk in range(D // 16):
                    rows[0][r, pl.ds(kk * 16, 16)] = jnp.zeros((16,), jnp.float32)

            for t in range(PER_SUB // CH):
                pltpu.sync_copy(rows[0],
                                acc_sh.at[pl.ds(s * PER_SUB + t * CH, CH)])
            plsc.subcore_barrier()

        for h in range(RPT // IDXH):
            with jax.named_scope("agg_idx_stage"):
                pltpu.sync_copy(src_hbm.at[pl.ds(base + h * IDXH, IDXH)], sidx_v)
                pltpu.sync_copy(dst_hbm.at[pl.ds(base + h * IDXH, IDXH)], didx_v)
            with jax.named_scope("agg_edges"):
                for b in range(NB):
                    pltpu.async_copy(g_hbm.at[sidx_v.at[b]], rows[b], gsem[b])

                @pl.loop(0, (IDXH - NB) // NB)
                def _(o):
                    t0 = o * NB
                    for b in range(NB):
                        t = t0 + b
                        pltpu.make_async_copy(
                            g_hbm.at[sidx_v.at[t]], rows[b], gsem[b]).wait()
                        pltpu.async_copy(
                            rows[b], acc_sh.at[didx_v.at[t]], ssem[b],
                            add=True).wait()
                        pltpu.async_copy(
                            g_hbm.at[sidx_v.at[t + NB]], rows[b], gsem[b])

                for b in range(NB):
                    t = IDXH - NB + b
                    pltpu.make_async_copy(
                        g_hbm.at[sidx_v.at[t]], rows[b], gsem[b]).wait()
                    pltpu.async_copy(
                        rows[b], acc_sh.at[didx_v.at[t]], ssem[b],
                        add=True).wait()

        with jax.named_scope("agg_drain"):
            plsc.subcore_barrier()
            pltpu.sync_copy(acc_sh.at[pl.ds(s * PER_SUB, PER_SUB)],
                            aggp_hbm.at[c, pl.ds(s * PER_SUB, PER_SUB)])

    return k(g, src2d, dst2d)


def _tc_bn_mm(x, g, b, W):
    def body(x_ref, g_ref, b_ref, w_ref, o_ref):
        xv = x_ref[...]
        mean = jnp.mean(xv, axis=0, keepdims=True)
        xc = xv - mean
        var = jnp.mean(xc * xc, axis=0, keepdims=True)
        hv = g_ref[...] * xc * lax.rsqrt(var + EPS) + b_ref[...]
        o_ref[...] = jnp.dot(hv, w_ref[...], preferred_element_type=jnp.float32)

    return pl.pallas_call(
        body, out_shape=jax.ShapeDtypeStruct((x.shape[0], W.shape[1]), jnp.float32)
    )(x, g, b, W)


def _tc_dis_scale(degp3, p):
    def body(d_ref, p_ref, dis_ref, g_ref):
        deg = d_ref[0] + d_ref[1] + 1.0
        dis = lax.rsqrt(deg)
        dis_ref[...] = dis
        g_ref[...] = dis[:N] * p_ref[...]

    return pl.pallas_call(
        body,
        out_shape=(
            jax.ShapeDtypeStruct((NPAD, 1), jnp.float32),
            jax.ShapeDtypeStruct((N, p.shape[1]), jnp.float32),
        ),
    )(degp3, p)


def _tc_mid(aggp, gself, dis, b, bng, bnb, W):
    def body(a_ref, gs_ref, d_ref, b_ref, bng_ref, bnb_ref, w_ref, o_ref):
        disn = d_ref[:N]
        h0 = disn * (a_ref[0, :N] + a_ref[1, :N] + gs_ref[...]) + b_ref[...]
        mean = jnp.mean(h0, axis=0, keepdims=True)
        hc = h0 - mean
        var = jnp.mean(hc * hc, axis=0, keepdims=True)
        hb = bng_ref[...] * hc * lax.rsqrt(var + EPS) + bnb_ref[...]
        hr = jnp.maximum(hb, 0.0)
        o_ref[...] = disn * jnp.dot(hr, w_ref[...],
                                    preferred_element_type=jnp.float32)

    return pl.pallas_call(
        body, out_shape=jax.ShapeDtypeStruct((N, W.shape[1]), jnp.float32)
    )(aggp, gself, dis, b, bng, bnb, W)


def _tc_final(aggp, gself, dis, b, bng, bnb, Wf, bf):
    def body(a_ref, gs_ref, d_ref, b_ref, bng_ref, bnb_ref, wf_ref, bf_ref, o_ref):
        disn = d_ref[:N]
        h0 = disn * (a_ref[0, :N] + a_ref[1, :N] + gs_ref[...]) + b_ref[...]
        mean = jnp.mean(h0, axis=0, keepdims=True)
        hc = h0 - mean
        var = jnp.mean(hc * hc, axis=0, keepdims=True)
        hb = bng_ref[...] * hc * lax.rsqrt(var + EPS) + bnb_ref[...]
        hr = jnp.maximum(hb, 0.0)
        o_ref[...] = lax.dot_general(
            hr, wf_ref[...], (((1,), (1,)), ((), ())),
            preferred_element_type=jnp.float32) + bf_ref[...]

    return pl.pallas_call(
        body, out_shape=jax.ShapeDtypeStruct((N, OUT), jnp.float32)
    )(aggp, gself, dis, b, bng, bnb, Wf, bf)


def kernel(x, bn_in_g, bn_in_b, W0, b0, bn0_g, bn0_b, W1, b1, bn1_g, bn1_b, Wf, bf, edge_index):
    pad = ROWS * CH - E
    pad_dst = N + jax.lax.rem(jnp.arange(pad, dtype=jnp.int32),
                              jnp.int32(NPAD - N - 1))
    src2d = jnp.concatenate(
        [edge_index[0], jnp.zeros((pad,), jnp.int32)]).reshape(ROWS, CH)
    dst2d = jnp.concatenate(
        [edge_index[1], pad_dst]).reshape(ROWS, CH)

    degp = _sc_degree(dst2d)
    p0 = _tc_bn_mm(x, bn_in_g, bn_in_b, W0)
    dis, g0 = _tc_dis_scale(degp.reshape(NCORE, NPAD, 1), p0)
    agg0 = _sc_aggregate(g0, src2d, dst2d)
    g1 = _tc_mid(agg0, g0, dis, b0, bn0_g, bn0_b, W1)
    agg1 = _sc_aggregate(g1, src2d, dst2d)
    out = _tc_final(agg1, g1, dis, b1, bn1_g, bn1_b, Wf, bf)
    return out


# speedup vs baseline: 3.2028x; 3.2028x over previous
import functools

import jax
import jax.numpy as jnp
from jax import lax
from jax.experimental import pallas as pl
from jax.experimental.pallas import tpu as pltpu
from jax.experimental.pallas import tpu_sc as plsc

N = 10000
E = 320000
D = 128
H = 128
OUT = 128
EPS = 1e-5

NPAD = 10240
CH = 128
ROWS = 2560
NSUB = 16
NCORE = 2
RPT = ROWS // (NSUB * NCORE)
PER_SUB = NPAD // NSUB
PAD_DST = NPAD - 2


def _sc_mesh():
    return plsc.VectorSubcoreMesh(core_axis_name="c", subcore_axis_name="s")


def _sc_degree(dst2d):
    @functools.partial(
        pl.kernel,
        out_type=jax.ShapeDtypeStruct((NCORE, NPAD), jnp.float32),
        mesh=_sc_mesh(),
        scratch_types=[
            pltpu.VMEM((1, CH), jnp.int32),
            pltpu.VMEM((CH,), jnp.float32),
            pltpu.VMEM((PER_SUB,), jnp.float32),
            pltpu.VMEM_SHARED((NPAD,), jnp.float32),
        ],
    )
    def k(dst_hbm, degp_hbm, idx_v, ones_v, zbuf_v, acc_sh):
        c = lax.axis_index("c")
        s = lax.axis_index("s")
        wid = c * NSUB + s
        for i in range(CH // 16):
            ones_v[pl.ds(i * 16, 16)] = jnp.ones((16,), jnp.float32)

        @pl.loop(0, PER_SUB // 16)
        def _(i):
            zbuf_v[pl.ds(i * 16, 16)] = jnp.zeros((16,), jnp.float32)

        pltpu.sync_copy(zbuf_v, acc_sh.at[pl.ds(s * PER_SUB, PER_SUB)])
        plsc.subcore_barrier()

        base = wid * RPT

        @pl.loop(0, RPT)
        def _(kk):
            pltpu.sync_copy(dst_hbm.at[pl.ds(base + kk, 1)], idx_v)
            pltpu.sync_copy(ones_v, acc_sh.at[idx_v.at[0]], add=True)

        plsc.subcore_barrier()
        pltpu.sync_copy(acc_sh.at[pl.ds(s * PER_SUB, PER_SUB)], zbuf_v)
        pltpu.sync_copy(zbuf_v, degp_hbm.at[c, pl.ds(s * PER_SUB, PER_SUB)])

    return k(dst2d)


NB = 2
IDXH = RPT // 2


def _sc_aggregate(g, src2d, dst2d):
    @functools.partial(
        pl.kernel,
        out_type=jax.ShapeDtypeStruct((NCORE, NPAD, D), jnp.float32),
        mesh=_sc_mesh(),
        scratch_types=[
            pltpu.VMEM((IDXH, CH), jnp.int32),
            pltpu.VMEM((IDXH, CH), jnp.int32),
        ]
        + [pltpu.VMEM((CH, D), jnp.float32) for _ in range(NB)]
        + [pltpu.VMEM_SHARED((NPAD, D), jnp.float32)]
        + [pltpu.SemaphoreType.DMA for _ in range(2 * NB)],
    )
    def k(g_hbm, src_hbm, dst_hbm, aggp_hbm, sidx_v, didx_v, *rest):
        rows = rest[:NB]
        acc_sh = rest[NB]
        gsem = rest[NB + 1:2 * NB + 1]
        ssem = rest[2 * NB + 1:3 * NB + 1]
        c = lax.axis_index("c")
        s = lax.axis_index("s")
        wid = c * NSUB + s
        base = wid * RPT

        with jax.named_scope("agg_zero"):
            @pl.loop(0, CH)
            def _(r):
                for kk in range(D // 16):
                    rows[0][r, pl.ds(kk * 16, 16)] = jnp.zeros((16,), jnp.float32)

            for t in range(PER_SUB // CH):
                pltpu.sync_copy(rows[0],
                                acc_sh.at[pl.ds(s * PER_SUB + t * CH, CH)])
            plsc.subcore_barrier()

        for h in range(RPT // IDXH):
            with jax.named_scope("agg_idx_stage"):
                pltpu.sync_copy(src_hbm.at[pl.ds(base + h * IDXH, IDXH)], sidx_v)
                pltpu.sync_copy(dst_hbm.at[pl.ds(base + h * IDXH, IDXH)], didx_v)
            with jax.named_scope("agg_edges"):
                for b in range(NB):
                    pltpu.async_copy(g_hbm.at[sidx_v.at[b]], rows[b], gsem[b])

                @pl.loop(0, (IDXH - NB) // NB)
                def _(o):
                    t0 = o * NB
                    for b in range(NB):
                        t = t0 + b
                        pltpu.make_async_copy(
                            g_hbm.at[sidx_v.at[t]], rows[b], gsem[b]).wait()
                        pltpu.async_copy(
                            rows[b], acc_sh.at[didx_v.at[t]], ssem[b],
                            add=True).wait()
                        pltpu.async_copy(
                            g_hbm.at[sidx_v.at[t + NB]], rows[b], gsem[b])

                for b in range(NB):
                    t = IDXH - NB + b
                    pltpu.make_async_copy(
                        g_hbm.at[sidx_v.at[t]], rows[b], gsem[b]).wait()
                    pltpu.async_copy(
                        rows[b], acc_sh.at[didx_v.at[t]], ssem[b],
                        add=True).wait()

        with jax.named_scope("agg_drain"):
            plsc.subcore_barrier()
            pltpu.sync_copy(acc_sh.at[pl.ds(s * PER_SUB, PER_SUB)],
                            aggp_hbm.at[c, pl.ds(s * PER_SUB, PER_SUB)])

    return k(g, src2d, dst2d)


def _tc_bn_mm(x, g, b, W):
    def body(x_ref, g_ref, b_ref, w_ref, o_ref):
        xv = x_ref[...]
        mean = jnp.mean(xv, axis=0, keepdims=True)
        xc = xv - mean
        var = jnp.mean(xc * xc, axis=0, keepdims=True)
        hv = g_ref[...] * xc * lax.rsqrt(var + EPS) + b_ref[...]
        o_ref[...] = jnp.dot(hv, w_ref[...], preferred_element_type=jnp.float32)

    return pl.pallas_call(
        body, out_shape=jax.ShapeDtypeStruct((x.shape[0], W.shape[1]), jnp.float32)
    )(x, g, b, W)


def _tc_dis_scale(degp3, p):
    def body(d_ref, p_ref, dis_ref, g_ref):
        deg = d_ref[0] + d_ref[1] + 1.0
        dis = lax.rsqrt(deg)
        dis_ref[...] = dis
        g_ref[...] = dis[:N] * p_ref[...]

    return pl.pallas_call(
        body,
        out_shape=(
            jax.ShapeDtypeStruct((NPAD, 1), jnp.float32),
            jax.ShapeDtypeStruct((N, p.shape[1]), jnp.float32),
        ),
    )(degp3, p)


def _tc_mid(aggp, gself, dis, b, bng, bnb, W):
    def body(a_ref, gs_ref, d_ref, b_ref, bng_ref, bnb_ref, w_ref, o_ref):
        disn = d_ref[:N]
        h0 = disn * (a_ref[0, :N] + a_ref[1, :N] + gs_ref[...]) + b_ref[...]
        mean = jnp.mean(h0, axis=0, keepdims=True)
        hc = h0 - mean
        var = jnp.mean(hc * hc, axis=0, keepdims=True)
        hb = bng_ref[...] * hc * lax.rsqrt(var + EPS) + bnb_ref[...]
        hr = jnp.maximum(hb, 0.0)
        o_ref[...] = disn * jnp.dot(hr, w_ref[...],
                                    preferred_element_type=jnp.float32)

    return pl.pallas_call(
        body, out_shape=jax.ShapeDtypeStruct((N, W.shape[1]), jnp.float32)
    )(aggp, gself, dis, b, bng, bnb, W)


def _tc_final(aggp, gself, dis, b, bng, bnb, Wf, bf):
    def body(a_ref, gs_ref, d_ref, b_ref, bng_ref, bnb_ref, wf_ref, bf_ref, o_ref):
        disn = d_ref[:N]
        h0 = disn * (a_ref[0, :N] + a_ref[1, :N] + gs_ref[...]) + b_ref[...]
        mean = jnp.mean(h0, axis=0, keepdims=True)
        hc = h0 - mean
        var = jnp.mean(hc * hc, axis=0, keepdims=True)
        hb = bng_ref[...] * hc * lax.rsqrt(var + EPS) + bnb_ref[...]
        hr = jnp.maximum(hb, 0.0)
        o_ref[...] = lax.dot_general(
            hr, wf_ref[...], (((1,), (1,)), ((), ())),
            preferred_element_type=jnp.float32) + bf_ref[...]

    return pl.pallas_call(
        body, out_shape=jax.ShapeDtypeStruct((N, OUT), jnp.float32)
    )(aggp, gself, dis, b, bng, bnb, Wf, bf)


def kernel(x, bn_in_g, bn_in_b, W0, b0, bn0_g, bn0_b, W1, b1, bn1_g, bn1_b, Wf, bf, edge_index):
    pad = ROWS * CH - E
    pad_dst = N + jax.lax.rem(jnp.arange(pad, dtype=jnp.int32),
                              jnp.int32(NPAD - N - 1))
    pad_src = jax.lax.rem(jnp.arange(pad, dtype=jnp.int32) * 37, jnp.int32(N))
    src2d = jnp.concatenate(
        [edge_index[0], pad_src]).reshape(ROWS, CH)
    dst2d = jnp.concatenate(
        [edge_index[1], pad_dst]).reshape(ROWS, CH)

    degp = _sc_degree(dst2d)
    p0 = _tc_bn_mm(x, bn_in_g, bn_in_b, W0)
    dis, g0 = _tc_dis_scale(degp.reshape(NCORE, NPAD, 1), p0)
    agg0 = _sc_aggregate(g0, src2d, dst2d)
    g1 = _tc_mid(agg0, g0, dis, b0, bn0_g, bn0_b, W1)
    agg1 = _sc_aggregate(g1, src2d, dst2d)
    out = _tc_final(agg1, g1, dis, b1, bn1_g, bn1_b, Wf, bf)
    return out


# speedup vs baseline: 3.7488x; 1.1705x over previous
import functools

import jax
import jax.numpy as jnp
from jax import lax
from jax.experimental import pallas as pl
from jax.experimental.pallas import tpu as pltpu
from jax.experimental.pallas import tpu_sc as plsc

N = 10000
E = 320000
D = 128
H = 128
OUT = 128
EPS = 1e-5

NPAD = 10240
CH = 64
ROWS = 5120
NSUB = 16
NCORE = 2
RPT = ROWS // (NSUB * NCORE)
PER_SUB = NPAD // NSUB
NB = 4
IDXH = RPT // 4
DSEM = 8


def _sc_mesh():
    return plsc.VectorSubcoreMesh(core_axis_name="c", subcore_axis_name="s")


def _sc_degree(dst2d):
    @functools.partial(
        pl.kernel,
        out_type=jax.ShapeDtypeStruct((NCORE, NPAD), jnp.float32),
        mesh=_sc_mesh(),
        scratch_types=[
            pltpu.VMEM((RPT, CH), jnp.int32),
            pltpu.VMEM((CH,), jnp.float32),
            pltpu.VMEM((PER_SUB,), jnp.float32),
            pltpu.VMEM_SHARED((NPAD,), jnp.float32),
        ]
        + [pltpu.SemaphoreType.DMA for _ in range(DSEM)],
    )
    def k(dst_hbm, degp_hbm, idx_v, ones_v, zbuf_v, acc_sh, *sems):
        c = lax.axis_index("c")
        s = lax.axis_index("s")
        wid = c * NSUB + s
        base = wid * RPT
        pltpu.sync_copy(dst_hbm.at[pl.ds(base, RPT)], idx_v)
        for i in range(CH // 16):
            ones_v[pl.ds(i * 16, 16)] = jnp.ones((16,), jnp.float32)

        @pl.loop(0, PER_SUB // 16)
        def _(i):
            zbuf_v[pl.ds(i * 16, 16)] = jnp.zeros((16,), jnp.float32)

        pltpu.sync_copy(zbuf_v, acc_sh.at[pl.ds(s * PER_SUB, PER_SUB)])
        plsc.subcore_barrier()

        @pl.loop(0, RPT // DSEM)
        def _(o):
            t0 = o * DSEM
            descs = [
                pltpu.async_copy(ones_v, acc_sh.at[idx_v.at[t0 + b]], sems[b],
                                 add=True)
                for b in range(DSEM)
            ]
            for dsc in descs:
                dsc.wait()

        plsc.subcore_barrier()
        pltpu.sync_copy(acc_sh.at[pl.ds(s * PER_SUB, PER_SUB)], zbuf_v)
        pltpu.sync_copy(zbuf_v, degp_hbm.at[c, pl.ds(s * PER_SUB, PER_SUB)])

    return k(dst2d)


def _sc_aggregate(g, src2d, dst2d):
    @functools.partial(
        pl.kernel,
        out_type=jax.ShapeDtypeStruct((NCORE, NPAD, D), jnp.float32),
        mesh=_sc_mesh(),
        scratch_types=[
            pltpu.VMEM((IDXH, CH), jnp.int32),
            pltpu.VMEM((IDXH, CH), jnp.int32),
        ]
        + [pltpu.VMEM((CH, D), jnp.float32) for _ in range(NB)]
        + [pltpu.VMEM_SHARED((NPAD, D), jnp.float32)]
        + [pltpu.SemaphoreType.DMA for _ in range(2 * NB)],
    )
    def k(g_hbm, src_hbm, dst_hbm, aggp_hbm, sidx_v, didx_v, *rest):
        rows = rest[:NB]
        acc_sh = rest[NB]
        gsem = rest[NB + 1:2 * NB + 1]
        ssem = rest[2 * NB + 1:3 * NB + 1]
        c = lax.axis_index("c")
        s = lax.axis_index("s")
        wid = c * NSUB + s
        base = wid * RPT

        with jax.named_scope("agg_zero"):
            @pl.loop(0, CH)
            def _(r):
                for kk in range(D // 16):
                    rows[0][r, pl.ds(kk * 16, 16)] = jnp.zeros((16,),
                                                               jnp.float32)

            for t in range(PER_SUB // CH):
                pltpu.sync_copy(rows[0],
                                acc_sh.at[pl.ds(s * PER_SUB + t * CH, CH)])
            plsc.subcore_barrier()

        for h in range(RPT // IDXH):
            with jax.named_scope("agg_idx_stage"):
                pltpu.sync_copy(src_hbm.at[pl.ds(base + h * IDXH, IDXH)],
                                sidx_v)
                pltpu.sync_copy(dst_hbm.at[pl.ds(base + h * IDXH, IDXH)],
                                didx_v)

            with jax.named_scope("agg_edges"):
                for b in range(NB):
                    pltpu.async_copy(g_hbm.at[sidx_v.at[b]], rows[b], gsem[b])

                @pl.loop(0, (IDXH - NB) // NB)
                def _(o):
                    t0 = o * NB
                    for b in range(NB):
                        t = t0 + b
                        pltpu.make_async_copy(
                            g_hbm.at[sidx_v.at[t]], rows[b], gsem[b]).wait()
                        pltpu.async_copy(
                            rows[b], acc_sh.at[didx_v.at[t]], ssem[b],
                            add=True).wait()
                        pltpu.async_copy(
                            g_hbm.at[sidx_v.at[t + NB]], rows[b], gsem[b])

                for b in range(NB):
                    t = IDXH - NB + b
                    pltpu.make_async_copy(
                        g_hbm.at[sidx_v.at[t]], rows[b], gsem[b]).wait()
                    pltpu.async_copy(
                        rows[b], acc_sh.at[didx_v.at[t]], ssem[b],
                        add=True).wait()

        with jax.named_scope("agg_drain"):
            plsc.subcore_barrier()
            pltpu.sync_copy(acc_sh.at[pl.ds(s * PER_SUB, PER_SUB)],
                            aggp_hbm.at[c, pl.ds(s * PER_SUB, PER_SUB)])

    return k(g, src2d, dst2d)


def _tc_bn_mm(x, g, b, W):
    def body(x_ref, g_ref, b_ref, w_ref, o_ref):
        xv = x_ref[...]
        mean = jnp.mean(xv, axis=0, keepdims=True)
        xc = xv - mean
        var = jnp.mean(xc * xc, axis=0, keepdims=True)
        hv = g_ref[...] * xc * lax.rsqrt(var + EPS) + b_ref[...]
        o_ref[...] = jnp.dot(hv, w_ref[...], preferred_element_type=jnp.float32)

    return pl.pallas_call(
        body, out_shape=jax.ShapeDtypeStruct((x.shape[0], W.shape[1]), jnp.float32)
    )(x, g, b, W)


def _tc_dis_scale(degp3, p):
    def body(d_ref, p_ref, dis_ref, g_ref):
        deg = d_ref[0] + d_ref[1] + 1.0
        dis = lax.rsqrt(deg)
        dis_ref[...] = dis
        g_ref[...] = dis[:N] * p_ref[...]

    return pl.pallas_call(
        body,
        out_shape=(
            jax.ShapeDtypeStruct((NPAD, 1), jnp.float32),
            jax.ShapeDtypeStruct((N, p.shape[1]), jnp.float32),
        ),
    )(degp3, p)


def _tc_mid(aggp, gself, dis, b, bng, bnb, W):
    def body(a_ref, gs_ref, d_ref, b_ref, bng_ref, bnb_ref, w_ref, o_ref):
        disn = d_ref[:N]
        h0 = disn * (a_ref[0, :N] + a_ref[1, :N] + gs_ref[...]) + b_ref[...]
        mean = jnp.mean(h0, axis=0, keepdims=True)
        hc = h0 - mean
        var = jnp.mean(hc * hc, axis=0, keepdims=True)
        hb = bng_ref[...] * hc * lax.rsqrt(var + EPS) + bnb_ref[...]
        hr = jnp.maximum(hb, 0.0)
        o_ref[...] = disn * jnp.dot(hr, w_ref[...],
                                    preferred_element_type=jnp.float32)

    return pl.pallas_call(
        body, out_shape=jax.ShapeDtypeStruct((N, W.shape[1]), jnp.float32)
    )(aggp, gself, dis, b, bng, bnb, W)


def _tc_final(aggp, gself, dis, b, bng, bnb, Wf, bf):
    def body(a_ref, gs_ref, d_ref, b_ref, bng_ref, bnb_ref, wf_ref, bf_ref, o_ref):
        disn = d_ref[:N]
        h0 = disn * (a_ref[0, :N] + a_ref[1, :N] + gs_ref[...]) + b_ref[...]
        mean = jnp.mean(h0, axis=0, keepdims=True)
        hc = h0 - mean
        var = jnp.mean(hc * hc, axis=0, keepdims=True)
        hb = bng_ref[...] * hc * lax.rsqrt(var + EPS) + bnb_ref[...]
        hr = jnp.maximum(hb, 0.0)
        o_ref[...] = lax.dot_general(
            hr, wf_ref[...], (((1,), (1,)), ((), ())),
            preferred_element_type=jnp.float32) + bf_ref[...]

    return pl.pallas_call(
        body, out_shape=jax.ShapeDtypeStruct((N, OUT), jnp.float32)
    )(aggp, gself, dis, b, bng, bnb, Wf, bf)


def kernel(x, bn_in_g, bn_in_b, W0, b0, bn0_g, bn0_b, W1, b1, bn1_g, bn1_b, Wf, bf, edge_index):
    pad = ROWS * CH - E
    pad_dst = N + jax.lax.rem(jnp.arange(pad, dtype=jnp.int32),
                              jnp.int32(NPAD - N - 1))
    pad_src = jax.lax.rem(jnp.arange(pad, dtype=jnp.int32) * 37, jnp.int32(N))
    src2d = jnp.concatenate(
        [edge_index[0], pad_src]).reshape(ROWS, CH)
    dst2d = jnp.concatenate(
        [edge_index[1], pad_dst]).reshape(ROWS, CH)

    degp = _sc_degree(dst2d)
    p0 = _tc_bn_mm(x, bn_in_g, bn_in_b, W0)
    dis, g0 = _tc_dis_scale(degp.reshape(NCORE, NPAD, 1), p0)
    agg0 = _sc_aggregate(g0, src2d, dst2d)
    g1 = _tc_mid(agg0, g0, dis, b0, bn0_g, bn0_b, W1)
    agg1 = _sc_aggregate(g1, src2d, dst2d)
    out = _tc_final(agg1, g1, dis, b1, bn1_g, bn1_b, Wf, bf)
    return out
